# Initial kernel scaffold; baseline (speedup 1.0000x reference)
import jax
import jax.numpy as jnp
from jax.experimental import pallas as pl


def kernel(x, semantic_table, positional_table):
    raise NotImplementedError("write your pallas kernel here")


# speedup vs baseline: 6.8440x; 6.8440x over previous
import functools

import jax
import jax.numpy as jnp
from jax import lax
from jax.experimental import pallas as pl
from jax.experimental.pallas import tpu as pltpu
from jax.experimental.pallas import tpu_sc as plsc

B = 4096
L = 2048
D = 32
V = 5

NC = 2
NS = 16
NW = NC * NS
RPW = B // NW

LANES = 16
IDX_PER_STREAM = 128
N_STREAMS = L // IDX_PER_STREAM


def _comb_body(sem_ref, pos_ref, out_ref):
    sem = sem_ref[...]
    pos = pos_ref[...]
    out_ref[...] = sem[:, None, :] + pos[None, :, :]


def _build_comb(semantic_table, positional_table):
    comb3 = pl.pallas_call(
        _comb_body,
        out_shape=jax.ShapeDtypeStruct((V, L, D), jnp.float32),
    )(semantic_table, positional_table)
    return comb3.reshape(V * L, D)


_mesh = plsc.VectorSubcoreMesh(core_axis_name="c", subcore_axis_name="s")


@functools.partial(
    pl.kernel,
    out_type=jax.ShapeDtypeStruct((B, L, D), jnp.float32),
    mesh=_mesh,
    scratch_types=[
        pltpu.VMEM((L,), jnp.int32),
        pltpu.VMEM((L,), jnp.int32),
        pltpu.VMEM((L, D), jnp.float32),
        pltpu.SemaphoreType.DMA,
    ],
    compiler_params=pltpu.CompilerParams(use_tc_tiling_on_sc=False),
)
def _sc_lookup(comb_hbm, x_hbm, out_hbm, xv, idxv, stage, gsem):
    wid = lax.axis_index("s") * NC + lax.axis_index("c")
    base = wid * RPW
    iota = lax.broadcasted_iota(jnp.int32, (LANES,), 0)

    def row_body(r, carry):
        row = base + r
        pltpu.sync_copy(x_hbm.at[row], xv)

        def chunk(j, carry2):
            for k in range(IDX_PER_STREAM // LANES):
                off = j * IDX_PER_STREAM + k * LANES
                x16 = xv[pl.ds(off, LANES)]
                idxv[pl.ds(off, LANES)] = x16 * L + (iota + off)
            return carry2

        lax.fori_loop(0, N_STREAMS, chunk, 0, unroll=False)

        copies = [
            pltpu.async_copy(
                comb_hbm.at[idxv.at[pl.ds(j * IDX_PER_STREAM, IDX_PER_STREAM)]],
                stage.at[pl.ds(j * IDX_PER_STREAM, IDX_PER_STREAM)],
                gsem,
            )
            for j in range(N_STREAMS)
        ]
        for cp in copies:
            cp.wait()
        pltpu.sync_copy(stage, out_hbm.at[row])
        return carry

    lax.fori_loop(0, RPW, row_body, 0, unroll=False)


def kernel(x, semantic_table, positional_table):
    comb = _build_comb(semantic_table, positional_table)
    return _sc_lookup(comb, x.astype(jnp.int32))


# speedup vs baseline: 7.6963x; 1.1245x over previous
import functools

import jax
import jax.numpy as jnp
from jax import lax
from jax.experimental import pallas as pl
from jax.experimental.pallas import tpu as pltpu
from jax.experimental.pallas import tpu_sc as plsc

B = 4096
L = 2048
D = 32
V = 5

NC = 2
NS = 16
NW = NC * NS
RPW = B // NW

LANES = 16
IDX_PER_STREAM = 128
N_STREAMS = L // IDX_PER_STREAM
HALF = L // 2


def _comb_body(sem_ref, pos_ref, out_ref):
    sem = sem_ref[...]
    pos = pos_ref[...]
    out_ref[...] = sem[:, None, :] + pos[None, :, :]


def _build_comb(semantic_table, positional_table):
    comb3 = pl.pallas_call(
        _comb_body,
        out_shape=jax.ShapeDtypeStruct((V, L, D), jnp.float32),
    )(semantic_table, positional_table)
    return comb3.reshape(V * L, D)


_mesh = plsc.VectorSubcoreMesh(core_axis_name="c", subcore_axis_name="s")


@functools.partial(
    pl.kernel,
    out_type=jax.ShapeDtypeStruct((B, L, D), jnp.float32),
    mesh=_mesh,
    scratch_types=[
        pltpu.VMEM((L,), jnp.int32),
        pltpu.VMEM((L,), jnp.int32),
        pltpu.VMEM((HALF, D), jnp.float32),
        pltpu.VMEM((HALF, D), jnp.float32),
        pltpu.VMEM_SHARED((V * L, D), jnp.float32),
        pltpu.SemaphoreType.DMA,
        pltpu.SemaphoreType.DMA,
        pltpu.SemaphoreType.DMA,
    ],
    compiler_params=pltpu.CompilerParams(use_tc_tiling_on_sc=False),
)
def _sc_lookup(comb_hbm, x_hbm, out_hbm, xv, idxv, stage0, stage1, comb_sp,
               gsem, osem0, osem1):
    c = lax.axis_index("c")
    s = lax.axis_index("s")
    wid = s * NC + c
    base = wid * RPW
    iota = lax.broadcasted_iota(jnp.int32, (LANES,), 0)

    @pl.when(s == 0)
    def _load_comb():
        pltpu.sync_copy(comb_hbm, comb_sp)

    plsc.subcore_barrier()

    stages = (stage0, stage1)
    osems = (osem0, osem1)

    def row_body(r, carry):
        row = base + r
        pltpu.sync_copy(x_hbm.at[row], xv)

        def chunk(j, carry2):
            for k in range(IDX_PER_STREAM // LANES):
                off = j * IDX_PER_STREAM + k * LANES
                x16 = xv[pl.ds(off, LANES)]
                idxv[pl.ds(off, LANES)] = x16 * L + (iota + off)
            return carry2

        lax.fori_loop(0, N_STREAMS, chunk, 0, unroll=False)

        for h in range(2):
            st, osem = stages[h], osems[h]

            @pl.when(r > 0)
            def _drain_prev():
                pltpu.make_async_copy(
                    st, out_hbm.at[base, pl.ds(h * HALF, HALF)], osem
                ).wait()

            copies = [
                pltpu.async_copy(
                    comb_sp.at[
                        idxv.at[pl.ds(h * HALF + j * IDX_PER_STREAM,
                                      IDX_PER_STREAM)]
                    ],
                    st.at[pl.ds(j * IDX_PER_STREAM, IDX_PER_STREAM)],
                    gsem,
                )
                for j in range(HALF // IDX_PER_STREAM)
            ]
            for cp in copies:
                cp.wait()
            pltpu.async_copy(st, out_hbm.at[row, pl.ds(h * HALF, HALF)], osem)
        return carry

    lax.fori_loop(0, RPW, row_body, 0, unroll=False)
    pltpu.make_async_copy(stage0, out_hbm.at[base, pl.ds(0, HALF)], osem0).wait()
    pltpu.make_async_copy(stage1, out_hbm.at[base, pl.ds(HALF, HALF)], osem1).wait()


def kernel(x, semantic_table, positional_table):
    comb = _build_comb(semantic_table, positional_table)
    return _sc_lookup(comb, x.astype(jnp.int32))
